# baseline (speedup 1.0000x reference)
import functools

import jax
import jax.numpy as jnp
from jax import lax
from jax.experimental import pallas as pl
from jax.experimental.pallas import tpu as pltpu
from jax.experimental.pallas import tpu_sc as plsc

B, S, D = 64, 2048, 512
TOPK_N = 8
SCALE = 1.0 / (S * D)

NSPLIT = 8
ROWS = S // NSPLIT
SPB = 2


def _mse_body(*refs):
    o_refs, l_refs, out_ref = refs[:NSPLIT], refs[NSPLIT:-1], refs[-1]
    i = pl.program_id(0)
    acc = jnp.zeros((SPB, 8, 128), jnp.float32)
    for o_ref, l_ref in zip(o_refs, l_refs):
        d = (o_ref[...] - l_ref[...]).reshape(SPB, -1, 8, 128)
        acc = acc + jnp.sum(d * d, axis=1)
    for s in range(SPB):
        out_ref[i * SPB + s] = jnp.sum(acc[s]) * SCALE


def _per_sample_mse(output, label):
    in_specs = [
        pl.BlockSpec((SPB, ROWS, D), lambda i, j=j: (i, j, 0))
        for j in range(NSPLIT)
    ]
    out_spec = pl.BlockSpec(memory_space=pltpu.SMEM)
    return pl.pallas_call(
        _mse_body,
        grid=(B // SPB,),
        in_specs=in_specs + in_specs,
        out_specs=out_spec,
        out_shape=jax.ShapeDtypeStruct((B,), jnp.float32),
    )(*([output] * NSPLIT), *([label] * NSPLIT))


def _vsort(x):
    k, _ = plsc.sort_key_val(x, x)
    return k


def _merge_top(a, b):
    return _vsort(jnp.maximum(a, jnp.flip(b, 0)))


@functools.cache
def _make_sc_top16():
    @functools.partial(
        pl.kernel,
        out_type=jax.ShapeDtypeStruct((16,), jnp.float32),
        mesh=plsc.VectorSubcoreMesh(
            core_axis_name="c", subcore_axis_name="s", num_cores=1),
        compiler_params=pltpu.CompilerParams(
            needs_layout_passes=False, skip_device_barrier=True,
            disable_bounds_checks=True, disable_semaphore_checks=True),
        scratch_types=[
            pltpu.VMEM((B,), jnp.float32),
            pltpu.VMEM((16,), jnp.float32),
        ],
    )
    def _sc_top16(losses_hbm, out_hbm, vals_v, out_v):
        cid = lax.axis_index("c")
        sid = lax.axis_index("s")

        @pl.when((cid == 0) & (sid == 0))
        def _():
            pltpu.sync_copy(losses_hbm, vals_v)
            s0 = _vsort(vals_v[pl.ds(0, 16)])
            s1 = _vsort(vals_v[pl.ds(16, 16)])
            s2 = _vsort(vals_v[pl.ds(32, 16)])
            s3 = _vsort(vals_v[pl.ds(48, 16)])
            top = _merge_top(_merge_top(s0, s1), _merge_top(s2, s3))
            out_v[...] = jnp.flip(top, 0)
            pltpu.sync_copy(out_v, out_hbm)

    return _sc_top16


def kernel(output, label):
    losses = _per_sample_mse(output, label)
    top16_desc = _make_sc_top16()(losses)
    return top16_desc[:TOPK_N]
